# baseline (speedup 1.0000x reference)
import jax
import jax.numpy as jnp
from jax.experimental import pallas as pl
from jax.experimental.pallas import tpu as pltpu

N_VARS = 32


def _desc_kernel(sv_ref, desc_ref, acc_ref):
    t = pl.program_id(0)
    d1 = pl.program_id(1)

    @pl.when((t == 0) & (d1 == 0))
    def _init():
        acc_ref[...] = jnp.zeros((8, N_VARS), jnp.float32)

    acc = acc_ref[...]
    for d0 in range(N_VARS):
        acc = acc + sv_ref[d0, 0]
    acc_ref[...] = acc

    @pl.when((t == pl.num_programs(0) - 1) & (d1 == pl.num_programs(1) - 1))
    def _finish():
        a = acc_ref[...]
        a = a[4:8, :] + a[0:4, :]
        a = a[2:4, :] + a[0:2, :]
        svs = a[1:2, :] + a[0:1, :]
        ii = jax.lax.broadcasted_iota(jnp.int32, (N_VARS, N_VARS), 0)
        jj = jax.lax.broadcasted_iota(jnp.int32, (N_VARS, N_VARS), 1)
        eye = ii == jj
        row = jnp.broadcast_to(svs, (N_VARS, N_VARS))
        col = jnp.sum(jnp.where(eye, row, 0.0), axis=1, keepdims=True)
        gt = (col > row) | ((col == row) & (ii < jj))
        rank = jnp.sum(gt.astype(jnp.int32), axis=0, keepdims=True)
        rank_row = jnp.broadcast_to(rank, (N_VARS, N_VARS))
        rank_col = jnp.sum(jnp.where(eye, rank_row, 0), axis=1, keepdims=True)
        onehot = (rank_col == jj).astype(jnp.int32)
        desc_ref[...] = jnp.sum(onehot * ii, axis=0, keepdims=True)


def _copy_kernel(desc_ref, x_ref, o_ref):
    del desc_ref
    o_ref[...] = x_ref[...]


def kernel(x, shuffle_vector):
    b, d, c, p = x.shape
    desc2 = pl.pallas_call(
        _desc_kernel,
        grid=(4, N_VARS),
        in_specs=[
            pl.BlockSpec((N_VARS, 1, 8, N_VARS), lambda t, d1: (0, d1, t, 0)),
        ],
        out_specs=pl.BlockSpec((1, N_VARS), lambda t, d1: (0, 0)),
        out_shape=jax.ShapeDtypeStruct((1, N_VARS), jnp.int32),
        scratch_shapes=[pltpu.VMEM((8, N_VARS), jnp.float32)],
    )(shuffle_vector)
    desc = desc2.reshape(N_VARS)

    rows = b * d
    x4 = x.reshape(rows, c, 1, p)
    rb = 256
    grid = (rows // rb, c)
    out4 = pl.pallas_call(
        _copy_kernel,
        grid_spec=pltpu.PrefetchScalarGridSpec(
            num_scalar_prefetch=1,
            grid=grid,
            in_specs=[
                pl.BlockSpec((rb, 1, 1, p), lambda r, ch, desc: (r, desc[ch], 0, 0)),
            ],
            out_specs=pl.BlockSpec((rb, 1, 1, p), lambda r, ch, desc: (r, ch, 0, 0)),
        ),
        out_shape=jax.ShapeDtypeStruct((rows, c, 1, p), jnp.float32),
    )(desc, x4)
    return out4.reshape(b, d, c, p)
